# baseline (speedup 1.0000x reference)
import dataclasses
import functools

import jax
import jax.numpy as jnp
from jax import lax
from jax.experimental import pallas as pl
from jax.experimental.pallas import tpu as pltpu
from jax.experimental.pallas import tpu_sc as plsc

_N = 10000
_E = 320000
_D = 128
_H = 128
_A = 16

_NC = 2
_NS = 16
_NW = _NC * _NS
_EPT = _E // _NW
_C = 100
_NCH = _EPT // _C
_G = 25
_NG = _NCH // _G
_SLAB = 624
_REM = _N - _NS * _SLAB

_mesh = plsc.VectorSubcoreMesh(core_axis_name="c", subcore_axis_name="s")

_sc_params = pltpu.CompilerParams()
if "needs_layout_passes" in pltpu.CompilerParams.__dataclass_fields__:
    _sc_params = dataclasses.replace(_sc_params, needs_layout_passes=False)


def _deg_body(dst_hbm, out_hbm, didx_v, deg_v):
    cid = lax.axis_index("c")
    sid = lax.axis_index("s")
    wid = cid * _NS + sid
    zeros = jnp.zeros((16,), jnp.float32)

    @pl.loop(0, _N, step=80)
    def _(i):
        for u in range(5):
            deg_v[pl.ds(i + u * 16, 16)] = zeros

    pltpu.sync_copy(dst_hbm.at[pl.ds(wid * _EPT, _EPT)], didx_v)
    ones = jnp.ones((16,), jnp.float32)

    @pl.loop(0, _EPT, step=80)
    def _(i):
        for u in range(5):
            idx = didx_v[pl.ds(i + u * 16, 16)]
            plsc.addupdate_scatter(deg_v, [idx], ones)

    pltpu.sync_copy(deg_v, out_hbm.at[wid])


@jax.jit
def _deg_call(dst1):
    return pl.kernel(
        _deg_body,
        out_type=jax.ShapeDtypeStruct((_NW, _N), jnp.float32),
        mesh=_mesh,
        compiler_params=_sc_params,
        scratch_types=[
            pltpu.VMEM((_EPT,), jnp.int32),
            pltpu.VMEM((_N,), jnp.float32),
        ],
    )(dst1)


def _edge_body(y_hbm, src_hbm, dst_hbm, out_hbm, sidx_v, didx_v, rows_v,
               rows1_v, agg_sh, sem, sem1):
    cid = lax.axis_index("c")
    sid = lax.axis_index("s")
    wid = cid * _NS + sid
    zeros = jnp.zeros((16,), jnp.float32)

    @pl.loop(0, _C)
    def _(r):
        @pl.loop(0, _D, step=16)
        def _(c0):
            rows_v[r, pl.ds(c0, 16)] = zeros

    base = pl.multiple_of(sid * _SLAB, 8)
    for m in range(_SLAB // _C):
        pltpu.sync_copy(rows_v, agg_sh.at[pl.ds(base + m * _C, _C)])
    pltpu.sync_copy(rows_v.at[pl.ds(0, _SLAB % _C)],
                    agg_sh.at[pl.ds(base + (_SLAB // _C) * _C, _SLAB % _C)])

    @pl.when(sid == 0)
    def _():
        pltpu.sync_copy(rows_v.at[pl.ds(0, _REM)],
                        agg_sh.at[pl.ds(_NS * _SLAB, _REM)])

    plsc.subcore_barrier()

    for g in range(_NG):
        pltpu.sync_copy(src_hbm.at[wid, g], sidx_v)
        pltpu.sync_copy(dst_hbm.at[wid, g], didx_v)
        pltpu.async_copy(y_hbm.at[sidx_v.at[0]], rows_v, sem)

        @pl.loop(0, (_G - 1) // 2)
        def _(t):
            j0 = 2 * t
            pltpu.async_copy(y_hbm.at[sidx_v.at[j0 + 1]], rows1_v, sem1)
            pltpu.make_async_copy(y_hbm.at[sidx_v.at[j0]], rows_v,
                                  sem).wait()
            pltpu.sync_copy(rows_v, agg_sh.at[didx_v.at[j0]], add=True)
            pltpu.async_copy(y_hbm.at[sidx_v.at[j0 + 2]], rows_v, sem)
            pltpu.make_async_copy(y_hbm.at[sidx_v.at[j0 + 1]], rows1_v,
                                  sem1).wait()
            pltpu.sync_copy(rows1_v, agg_sh.at[didx_v.at[j0 + 1]], add=True)

        pltpu.make_async_copy(y_hbm.at[sidx_v.at[_G - 1]], rows_v,
                              sem).wait()
        pltpu.sync_copy(rows_v, agg_sh.at[didx_v.at[_G - 1]], add=True)

    plsc.subcore_barrier()
    pltpu.sync_copy(agg_sh.at[pl.ds(base, _SLAB)],
                    out_hbm.at[cid, pl.ds(base, _SLAB)])

    @pl.when(sid == 0)
    def _():
        pltpu.sync_copy(agg_sh.at[pl.ds(_NS * _SLAB, _REM)],
                        out_hbm.at[cid, pl.ds(_NS * _SLAB, _REM)])


@jax.jit
def _edge_call(y, src3, dst3):
    return pl.kernel(
        _edge_body,
        out_type=jax.ShapeDtypeStruct((_NC, _N, _D), jnp.float32),
        mesh=_mesh,
        scratch_types=[
            pltpu.VMEM((_G, _C), jnp.int32),
            pltpu.VMEM((_G, _C), jnp.int32),
            pltpu.VMEM((_C, _D), jnp.float32),
            pltpu.VMEM((_C, _D), jnp.float32),
            pltpu.VMEM_SHARED((_N, _D), jnp.float32),
            pltpu.SemaphoreType.DMA,
            pltpu.SemaphoreType.DMA,
        ],
    )(y, src3, dst3)


def _xw_body(x_ref, w_ref, xw_ref):
    xw_ref[...] = jnp.dot(x_ref[...], w_ref[...],
                          preferred_element_type=jnp.float32)


@jax.jit
def _xw_call(x, w):
    blk = 1000
    return pl.pallas_call(
        _xw_body,
        grid=(_N // blk,),
        in_specs=[
            pl.BlockSpec((blk, _D), lambda i: (i, 0)),
            pl.BlockSpec((_D, _H), lambda i: (0, 0)),
        ],
        out_specs=pl.BlockSpec((blk, _H), lambda i: (i, 0)),
        out_shape=jax.ShapeDtypeStruct((_N, _H), jnp.float32),
    )(x, w)


def _scale_body(degp_ref, xw_ref, y_ref, dis_ref):
    deg = jnp.sum(degp_ref[...], axis=0, keepdims=True) + 1.0
    dis8 = jnp.broadcast_to(lax.rsqrt(deg), (8, deg.shape[1]))
    dis_col = jnp.transpose(dis8)[:, :1]
    dis_ref[...] = dis_col
    y_ref[...] = xw_ref[...] * dis_col


@jax.jit
def _scale_call(deg_part, xw):
    return pl.pallas_call(
        _scale_body,
        out_shape=[
            jax.ShapeDtypeStruct((_N, _H), jnp.float32),
            jax.ShapeDtypeStruct((_N, 1), jnp.float32),
        ],
    )(deg_part, xw)


def _sigmoid(u):
    return 1.0 / (1.0 + jnp.exp(-u))


def _post_body(aggp_ref, y_ref, dis_ref, h_ref, wih_ref, whh_ref, bih_ref,
               bhh_ref, wfc_ref, bgcn_ref, bfc_ref, q_ref, h_out_ref):
    aggp = aggp_ref[...]
    y = y_ref[...]
    agg = aggp[0] + aggp[1] + y
    x2 = jnp.maximum(agg * dis_ref[...] + bgcn_ref[...], 0.0)
    h = h_ref[...]
    gi = jnp.dot(x2, wih_ref[...], preferred_element_type=jnp.float32) \
        + bih_ref[...]
    gh = jnp.dot(h, whh_ref[...], preferred_element_type=jnp.float32) \
        + bhh_ref[...]
    r = _sigmoid(gi[:, :_H] + gh[:, :_H])
    z = _sigmoid(gi[:, _H:2 * _H] + gh[:, _H:2 * _H])
    n = jnp.tanh(gi[:, 2 * _H:] + r * gh[:, 2 * _H:])
    hn = (1.0 - z) * n + z * h
    h_out_ref[...] = hn
    q_ref[...] = jnp.dot(hn, wfc_ref[...],
                         preferred_element_type=jnp.float32) + bfc_ref[...]


@jax.jit
def _post_call(agg_part, y, dis_col, h_in, wih_t, whh_t, bih, bhh, wfc_t,
               bgcn, bfc):
    blk = 1000
    grid = (_N // blk,)
    return pl.pallas_call(
        _post_body,
        grid=grid,
        in_specs=[
            pl.BlockSpec((_NC, blk, _D), lambda i: (0, i, 0)),
            pl.BlockSpec((blk, _H), lambda i: (i, 0)),
            pl.BlockSpec((blk, 1), lambda i: (i, 0)),
            pl.BlockSpec((blk, _H), lambda i: (i, 0)),
            pl.BlockSpec((_H, 3 * _H), lambda i: (0, 0)),
            pl.BlockSpec((_H, 3 * _H), lambda i: (0, 0)),
            pl.BlockSpec((1, 3 * _H), lambda i: (0, 0)),
            pl.BlockSpec((1, 3 * _H), lambda i: (0, 0)),
            pl.BlockSpec((_H, _A), lambda i: (0, 0)),
            pl.BlockSpec((1, _D), lambda i: (0, 0)),
            pl.BlockSpec((1, _A), lambda i: (0, 0)),
        ],
        out_specs=[
            pl.BlockSpec((blk, _A), lambda i: (i, 0)),
            pl.BlockSpec((blk, _H), lambda i: (i, 0)),
        ],
        out_shape=[
            jax.ShapeDtypeStruct((_N, _A), jnp.float32),
            jax.ShapeDtypeStruct((_N, _H), jnp.float32),
        ],
    )(agg_part, y, dis_col, h_in, wih_t, whh_t, bih, bhh, wfc_t, bgcn, bfc)


def kernel(inputs, hidden_state, edge_index, W_gcn, b_gcn, W_ih, W_hh, b_ih,
           b_hh, W_fc2, b_fc2):
    src3 = edge_index[0].reshape(_NW, _NG, _G, _C)
    dst1 = edge_index[1]
    dst3 = dst1.reshape(_NW, _NG, _G, _C)

    deg_part = _deg_call(dst1)
    xw = _xw_call(inputs, W_gcn)
    y, dis_col = _scale_call(deg_part, xw)
    agg_part = _edge_call(y, src3, dst3)
    q, h = _post_call(
        agg_part, y, dis_col, hidden_state.reshape(_N, _H),
        W_ih.T, W_hh.T, b_ih.reshape(1, -1), b_hh.reshape(1, -1),
        W_fc2.T, b_gcn.reshape(1, -1), b_fc2.reshape(1, -1))
    return q, h


# speedup vs baseline: 1.0521x; 1.0521x over previous
import dataclasses
import functools

import jax
import jax.numpy as jnp
from jax import lax
from jax.experimental import pallas as pl
from jax.experimental.pallas import tpu as pltpu
from jax.experimental.pallas import tpu_sc as plsc

_N = 10000
_E = 320000
_D = 128
_H = 128
_A = 16

_NC = 2
_NS = 16
_NW = _NC * _NS
_EPT = _E // _NW
_C = 125
_NCH = _EPT // _C
_G = 20
_NG = _NCH // _G
_SLAB = 624
_REM = _N - _NS * _SLAB

_mesh = plsc.VectorSubcoreMesh(core_axis_name="c", subcore_axis_name="s")

_sc_params = pltpu.CompilerParams()
if "needs_layout_passes" in pltpu.CompilerParams.__dataclass_fields__:
    _sc_params = dataclasses.replace(_sc_params, needs_layout_passes=False)


def _deg_body(dst_hbm, out_hbm, didx_v, deg_v):
    cid = lax.axis_index("c")
    sid = lax.axis_index("s")
    wid = cid * _NS + sid
    zeros = jnp.zeros((16,), jnp.float32)

    @pl.loop(0, _N, step=16)
    def _(i):
        deg_v[pl.ds(i, 16)] = zeros

    pltpu.sync_copy(dst_hbm.at[pl.ds(wid * _EPT, _EPT)], didx_v)
    ones = jnp.ones((16,), jnp.float32)

    @pl.loop(0, _EPT, step=16)
    def _(i):
        idx = didx_v[pl.ds(i, 16)]
        plsc.addupdate_scatter(deg_v, [idx], ones)

    pltpu.sync_copy(deg_v, out_hbm.at[wid])


@jax.jit
def _deg_call(dst1):
    return pl.kernel(
        _deg_body,
        out_type=jax.ShapeDtypeStruct((_NW, _N), jnp.float32),
        mesh=_mesh,
        compiler_params=_sc_params,
        scratch_types=[
            pltpu.VMEM((_EPT,), jnp.int32),
            pltpu.VMEM((_N,), jnp.float32),
        ],
    )(dst1)


def _edge_body(y_hbm, src_hbm, dst_hbm, out_hbm, sidx_v, didx_v, rows_v,
               rows1_v, agg_sh, sem, sem1):
    cid = lax.axis_index("c")
    sid = lax.axis_index("s")
    wid = cid * _NS + sid
    zeros = jnp.zeros((16,), jnp.float32)

    @pl.loop(0, _C)
    def _(r):
        @pl.loop(0, _D, step=16)
        def _(c0):
            rows_v[r, pl.ds(c0, 16)] = zeros

    base = pl.multiple_of(sid * _SLAB, 8)
    for m in range(_SLAB // _C):
        pltpu.sync_copy(rows_v, agg_sh.at[pl.ds(base + m * _C, _C)])
    pltpu.sync_copy(rows_v.at[pl.ds(0, _SLAB % _C)],
                    agg_sh.at[pl.ds(base + (_SLAB // _C) * _C, _SLAB % _C)])

    @pl.when(sid == 0)
    def _():
        pltpu.sync_copy(rows_v.at[pl.ds(0, _REM)],
                        agg_sh.at[pl.ds(_NS * _SLAB, _REM)])

    plsc.subcore_barrier()

    for g in range(_NG):
        pltpu.sync_copy(src_hbm.at[wid, g], sidx_v)
        pltpu.sync_copy(dst_hbm.at[wid, g], didx_v)
        pltpu.async_copy(y_hbm.at[sidx_v.at[0]], rows_v, sem)

        @pl.loop(0, (_G - 2) // 2 if _G % 2 == 0 else (_G - 1) // 2)
        def _(t):
            j0 = 2 * t
            pltpu.async_copy(y_hbm.at[sidx_v.at[j0 + 1]], rows1_v, sem1)
            pltpu.make_async_copy(y_hbm.at[sidx_v.at[j0]], rows_v,
                                  sem).wait()
            pltpu.sync_copy(rows_v, agg_sh.at[didx_v.at[j0]], add=True)
            pltpu.async_copy(y_hbm.at[sidx_v.at[j0 + 2]], rows_v, sem)
            pltpu.make_async_copy(y_hbm.at[sidx_v.at[j0 + 1]], rows1_v,
                                  sem1).wait()
            pltpu.sync_copy(rows1_v, agg_sh.at[didx_v.at[j0 + 1]], add=True)

        if _G % 2 == 0:
            pltpu.async_copy(y_hbm.at[sidx_v.at[_G - 1]], rows1_v, sem1)
            pltpu.make_async_copy(y_hbm.at[sidx_v.at[_G - 2]], rows_v,
                                  sem).wait()
            pltpu.sync_copy(rows_v, agg_sh.at[didx_v.at[_G - 2]], add=True)
            pltpu.make_async_copy(y_hbm.at[sidx_v.at[_G - 1]], rows1_v,
                                  sem1).wait()
            pltpu.sync_copy(rows1_v, agg_sh.at[didx_v.at[_G - 1]], add=True)
        else:
            pltpu.make_async_copy(y_hbm.at[sidx_v.at[_G - 1]], rows_v,
                                  sem).wait()
            pltpu.sync_copy(rows_v, agg_sh.at[didx_v.at[_G - 1]], add=True)

    plsc.subcore_barrier()
    pltpu.sync_copy(agg_sh.at[pl.ds(base, _SLAB)],
                    out_hbm.at[cid, pl.ds(base, _SLAB)])

    @pl.when(sid == 0)
    def _():
        pltpu.sync_copy(agg_sh.at[pl.ds(_NS * _SLAB, _REM)],
                        out_hbm.at[cid, pl.ds(_NS * _SLAB, _REM)])


@jax.jit
def _edge_call(y, src3, dst3):
    return pl.kernel(
        _edge_body,
        out_type=jax.ShapeDtypeStruct((_NC, _N, _D), jnp.float32),
        mesh=_mesh,
        scratch_types=[
            pltpu.VMEM((_G, _C), jnp.int32),
            pltpu.VMEM((_G, _C), jnp.int32),
            pltpu.VMEM((_C, _D), jnp.float32),
            pltpu.VMEM((_C, _D), jnp.float32),
            pltpu.VMEM_SHARED((_N, _D), jnp.float32),
            pltpu.SemaphoreType.DMA,
            pltpu.SemaphoreType.DMA,
        ],
    )(y, src3, dst3)


def _pre_body(degp_ref, x_ref, w_ref, y_ref, dis_ref):
    deg = jnp.sum(degp_ref[...], axis=0, keepdims=True) + 1.0
    dis8 = jnp.broadcast_to(lax.rsqrt(deg), (8, deg.shape[1]))
    dis_col = jnp.transpose(dis8)[:, :1]
    dis_ref[...] = dis_col
    y_ref[...] = jnp.dot(x_ref[...], w_ref[...],
                         preferred_element_type=jnp.float32) * dis_col


@jax.jit
def _pre_call(deg_part, x, w):
    return pl.pallas_call(
        _pre_body,
        out_shape=[
            jax.ShapeDtypeStruct((_N, _H), jnp.float32),
            jax.ShapeDtypeStruct((_N, 1), jnp.float32),
        ],
    )(deg_part, x, w)


def _sigmoid(u):
    return 1.0 / (1.0 + jnp.exp(-u))


def _post_body(aggp_ref, y_ref, dis_ref, h_ref, wih_ref, whh_ref, bih_ref,
               bhh_ref, wfc_ref, bgcn_ref, bfc_ref, q_ref, h_out_ref):
    aggp = aggp_ref[...]
    y = y_ref[...]
    agg = aggp[0] + aggp[1] + y
    x2 = jnp.maximum(agg * dis_ref[...] + bgcn_ref[...], 0.0)
    h = h_ref[...]
    gi = jnp.dot(x2, wih_ref[...], preferred_element_type=jnp.float32) \
        + bih_ref[...]
    gh = jnp.dot(h, whh_ref[...], preferred_element_type=jnp.float32) \
        + bhh_ref[...]
    r = _sigmoid(gi[:, :_H] + gh[:, :_H])
    z = _sigmoid(gi[:, _H:2 * _H] + gh[:, _H:2 * _H])
    n = jnp.tanh(gi[:, 2 * _H:] + r * gh[:, 2 * _H:])
    hn = (1.0 - z) * n + z * h
    h_out_ref[...] = hn
    q_ref[...] = jnp.dot(hn, wfc_ref[...],
                         preferred_element_type=jnp.float32) + bfc_ref[...]


@jax.jit
def _post_call(agg_part, y, dis_col, h_in, wih_t, whh_t, bih, bhh, wfc_t,
               bgcn, bfc):
    blk = 1000
    grid = (_N // blk,)
    return pl.pallas_call(
        _post_body,
        grid=grid,
        in_specs=[
            pl.BlockSpec((_NC, blk, _D), lambda i: (0, i, 0)),
            pl.BlockSpec((blk, _H), lambda i: (i, 0)),
            pl.BlockSpec((blk, 1), lambda i: (i, 0)),
            pl.BlockSpec((blk, _H), lambda i: (i, 0)),
            pl.BlockSpec((_H, 3 * _H), lambda i: (0, 0)),
            pl.BlockSpec((_H, 3 * _H), lambda i: (0, 0)),
            pl.BlockSpec((1, 3 * _H), lambda i: (0, 0)),
            pl.BlockSpec((1, 3 * _H), lambda i: (0, 0)),
            pl.BlockSpec((_H, _A), lambda i: (0, 0)),
            pl.BlockSpec((1, _D), lambda i: (0, 0)),
            pl.BlockSpec((1, _A), lambda i: (0, 0)),
        ],
        out_specs=[
            pl.BlockSpec((blk, _A), lambda i: (i, 0)),
            pl.BlockSpec((blk, _H), lambda i: (i, 0)),
        ],
        out_shape=[
            jax.ShapeDtypeStruct((_N, _A), jnp.float32),
            jax.ShapeDtypeStruct((_N, _H), jnp.float32),
        ],
    )(agg_part, y, dis_col, h_in, wih_t, whh_t, bih, bhh, wfc_t, bgcn, bfc)


def kernel(inputs, hidden_state, edge_index, W_gcn, b_gcn, W_ih, W_hh, b_ih,
           b_hh, W_fc2, b_fc2):
    src3 = edge_index[0].reshape(_NW, _NG, _G, _C)
    dst1 = edge_index[1]
    dst3 = dst1.reshape(_NW, _NG, _G, _C)

    deg_part = _deg_call(dst1)
    y, dis_col = _pre_call(deg_part, inputs, W_gcn)
    agg_part = _edge_call(y, src3, dst3)
    q, h = _post_call(
        agg_part, y, dis_col, hidden_state.reshape(_N, _H),
        W_ih.T, W_hh.T, b_ih.reshape(1, -1), b_hh.reshape(1, -1),
        W_fc2.T, b_gcn.reshape(1, -1), b_fc2.reshape(1, -1))
    return q, h


# speedup vs baseline: 1.0525x; 1.0004x over previous
import dataclasses

import jax
import jax.numpy as jnp
from jax import lax
from jax.experimental import pallas as pl
from jax.experimental.pallas import tpu as pltpu
from jax.experimental.pallas import tpu_sc as plsc

_N = 10000
_E = 320000
_D = 128
_H = 128
_A = 16

_NC = 2
_NS = 16
_NW = _NC * _NS
_EPT = _E // _NW
_C = 125
_NCH = _EPT // _C
_G = 20
_NG = _NCH // _G
_SLAB = 624
_REM = _N - _NS * _SLAB

_mesh = plsc.VectorSubcoreMesh(core_axis_name="c", subcore_axis_name="s")

_sc_params = pltpu.CompilerParams()
if "needs_layout_passes" in pltpu.CompilerParams.__dataclass_fields__:
    _sc_params = dataclasses.replace(_sc_params, needs_layout_passes=False)


def _deg_body(dst_hbm, out_hbm, didx_v, deg_v):
    cid = lax.axis_index("c")
    sid = lax.axis_index("s")
    wid = cid * _NS + sid
    zeros = jnp.zeros((16,), jnp.float32)

    @pl.loop(0, _N, step=16)
    def _(i):
        deg_v[pl.ds(i, 16)] = zeros

    pltpu.sync_copy(dst_hbm.at[pl.ds(wid * _EPT, _EPT)], didx_v)
    ones = jnp.ones((16,), jnp.float32)

    @pl.loop(0, _EPT, step=16)
    def _(i):
        idx = didx_v[pl.ds(i, 16)]
        plsc.addupdate_scatter(deg_v, [idx], ones)

    pltpu.sync_copy(deg_v, out_hbm.at[wid])


@jax.jit
def _deg_call(dst1):
    return pl.kernel(
        _deg_body,
        out_type=jax.ShapeDtypeStruct((_NW, _N), jnp.float32),
        mesh=_mesh,
        compiler_params=_sc_params,
        scratch_types=[
            pltpu.VMEM((_EPT,), jnp.int32),
            pltpu.VMEM((_N,), jnp.float32),
        ],
    )(dst1)


def _edge_body(y_hbm, src_hbm, dst_hbm, out_hbm, sidx_v, didx_v, rows_v,
               rows1_v, agg_sh, sem, sem1):
    cid = lax.axis_index("c")
    sid = lax.axis_index("s")
    wid = cid * _NS + sid
    zeros = jnp.zeros((16,), jnp.float32)

    @pl.loop(0, _C)
    def _(r):
        @pl.loop(0, _D, step=16)
        def _(c0):
            rows_v[r, pl.ds(c0, 16)] = zeros

    base = pl.multiple_of(sid * _SLAB, 8)
    for m in range(_SLAB // _C):
        pltpu.sync_copy(rows_v, agg_sh.at[pl.ds(base + m * _C, _C)])
    pltpu.sync_copy(rows_v.at[pl.ds(0, _SLAB % _C)],
                    agg_sh.at[pl.ds(base + (_SLAB // _C) * _C, _SLAB % _C)])

    @pl.when(sid == 0)
    def _():
        pltpu.sync_copy(rows_v.at[pl.ds(0, _REM)],
                        agg_sh.at[pl.ds(_NS * _SLAB, _REM)])

    plsc.subcore_barrier()

    for g in range(_NG):
        pltpu.sync_copy(src_hbm.at[wid, g], sidx_v)
        pltpu.sync_copy(dst_hbm.at[wid, g], didx_v)
        pltpu.async_copy(y_hbm.at[sidx_v.at[0]], rows_v, sem)

        @pl.loop(0, (_G - 2) // 2 if _G % 2 == 0 else (_G - 1) // 2)
        def _(t):
            j0 = 2 * t
            pltpu.async_copy(y_hbm.at[sidx_v.at[j0 + 1]], rows1_v, sem1)
            pltpu.make_async_copy(y_hbm.at[sidx_v.at[j0]], rows_v,
                                  sem).wait()
            pltpu.sync_copy(rows_v, agg_sh.at[didx_v.at[j0]], add=True)
            pltpu.async_copy(y_hbm.at[sidx_v.at[j0 + 2]], rows_v, sem)
            pltpu.make_async_copy(y_hbm.at[sidx_v.at[j0 + 1]], rows1_v,
                                  sem1).wait()
            pltpu.sync_copy(rows1_v, agg_sh.at[didx_v.at[j0 + 1]], add=True)

        if _G % 2 == 0:
            pltpu.async_copy(y_hbm.at[sidx_v.at[_G - 1]], rows1_v, sem1)
            pltpu.make_async_copy(y_hbm.at[sidx_v.at[_G - 2]], rows_v,
                                  sem).wait()
            pltpu.sync_copy(rows_v, agg_sh.at[didx_v.at[_G - 2]], add=True)
            pltpu.make_async_copy(y_hbm.at[sidx_v.at[_G - 1]], rows1_v,
                                  sem1).wait()
            pltpu.sync_copy(rows1_v, agg_sh.at[didx_v.at[_G - 1]], add=True)
        else:
            pltpu.make_async_copy(y_hbm.at[sidx_v.at[_G - 1]], rows_v,
                                  sem).wait()
            pltpu.sync_copy(rows_v, agg_sh.at[didx_v.at[_G - 1]], add=True)

    plsc.subcore_barrier()
    pltpu.sync_copy(agg_sh.at[pl.ds(base, _SLAB)],
                    out_hbm.at[cid, pl.ds(base, _SLAB)])

    @pl.when(sid == 0)
    def _():
        pltpu.sync_copy(agg_sh.at[pl.ds(_NS * _SLAB, _REM)],
                        out_hbm.at[cid, pl.ds(_NS * _SLAB, _REM)])


@jax.jit
def _edge_call(y, src3, dst3):
    return pl.kernel(
        _edge_body,
        out_type=jax.ShapeDtypeStruct((_NC, _N, _D), jnp.float32),
        mesh=_mesh,
        scratch_types=[
            pltpu.VMEM((_G, _C), jnp.int32),
            pltpu.VMEM((_G, _C), jnp.int32),
            pltpu.VMEM((_C, _D), jnp.float32),
            pltpu.VMEM((_C, _D), jnp.float32),
            pltpu.VMEM_SHARED((_N, _D), jnp.float32),
            pltpu.SemaphoreType.DMA,
            pltpu.SemaphoreType.DMA,
        ],
    )(y, src3, dst3)


def _pre_body(degp_ref, x_ref, w_ref, y_ref, dis_ref):
    deg = jnp.sum(degp_ref[...], axis=0, keepdims=True) + 1.0
    dis8 = jnp.broadcast_to(lax.rsqrt(deg), (8, deg.shape[1]))
    dis_col = jnp.transpose(dis8)[:, :1]
    dis_ref[...] = dis_col
    y_ref[...] = jnp.dot(x_ref[...], w_ref[...],
                         preferred_element_type=jnp.float32) * dis_col


@jax.jit
def _pre_call(deg_part, x, w):
    return pl.pallas_call(
        _pre_body,
        out_shape=[
            jax.ShapeDtypeStruct((_N, _H), jnp.float32),
            jax.ShapeDtypeStruct((_N, 1), jnp.float32),
        ],
    )(deg_part, x, w)


def _sigmoid(u):
    return 1.0 / (1.0 + jnp.exp(-u))


def _post_body(aggp_ref, y_ref, dis_ref, h_ref, wih_ref, whh_ref, bih_ref,
               bhh_ref, wfc_ref, bgcn_ref, bfc_ref, q_ref, h_out_ref):
    aggp = aggp_ref[...]
    y = y_ref[...]
    agg = aggp[0] + aggp[1] + y
    x2 = jnp.maximum(agg * dis_ref[...] + bgcn_ref[...], 0.0)
    h = h_ref[...]
    gi = jnp.dot(x2, wih_ref[...], preferred_element_type=jnp.float32) \
        + bih_ref[...]
    gh = jnp.dot(h, whh_ref[...], preferred_element_type=jnp.float32) \
        + bhh_ref[...]
    r = _sigmoid(gi[:, :_H] + gh[:, :_H])
    z = _sigmoid(gi[:, _H:2 * _H] + gh[:, _H:2 * _H])
    n = jnp.tanh(gi[:, 2 * _H:] + r * gh[:, 2 * _H:])
    hn = (1.0 - z) * n + z * h
    h_out_ref[...] = hn
    q_ref[...] = jnp.dot(hn, wfc_ref[...],
                         preferred_element_type=jnp.float32) + bfc_ref[...]


@jax.jit
def _post_call(agg_part, y, dis_col, h_in, wih_t, whh_t, bih, bhh, wfc_t,
               bgcn, bfc):
    blk = 1000
    grid = (_N // blk,)
    return pl.pallas_call(
        _post_body,
        grid=grid,
        in_specs=[
            pl.BlockSpec((_NC, blk, _D), lambda i: (0, i, 0)),
            pl.BlockSpec((blk, _H), lambda i: (i, 0)),
            pl.BlockSpec((blk, 1), lambda i: (i, 0)),
            pl.BlockSpec((blk, _H), lambda i: (i, 0)),
            pl.BlockSpec((_H, 3 * _H), lambda i: (0, 0)),
            pl.BlockSpec((_H, 3 * _H), lambda i: (0, 0)),
            pl.BlockSpec((1, 3 * _H), lambda i: (0, 0)),
            pl.BlockSpec((1, 3 * _H), lambda i: (0, 0)),
            pl.BlockSpec((_H, _A), lambda i: (0, 0)),
            pl.BlockSpec((1, _D), lambda i: (0, 0)),
            pl.BlockSpec((1, _A), lambda i: (0, 0)),
        ],
        out_specs=[
            pl.BlockSpec((blk, _A), lambda i: (i, 0)),
            pl.BlockSpec((blk, _H), lambda i: (i, 0)),
        ],
        out_shape=[
            jax.ShapeDtypeStruct((_N, _A), jnp.float32),
            jax.ShapeDtypeStruct((_N, _H), jnp.float32),
        ],
    )(agg_part, y, dis_col, h_in, wih_t, whh_t, bih, bhh, wfc_t, bgcn, bfc)


def kernel(inputs, hidden_state, edge_index, W_gcn, b_gcn, W_ih, W_hh, b_ih,
           b_hh, W_fc2, b_fc2):
    src3 = edge_index[0].reshape(_NW, _NG, _G, _C)
    dst1 = edge_index[1]
    dst3 = dst1.reshape(_NW, _NG, _G, _C)

    deg_part = _deg_call(dst1)
    y, dis_col = _pre_call(deg_part, inputs, W_gcn)
    agg_part = _edge_call(y, src3, dst3)
    q, h = _post_call(
        agg_part, y, dis_col, hidden_state.reshape(_N, _H),
        W_ih.T, W_hh.T, b_ih.reshape(1, -1), b_hh.reshape(1, -1),
        W_fc2.T, b_gcn.reshape(1, -1), b_fc2.reshape(1, -1))
    return q, h
